# baseline (speedup 1.0000x reference)
import jax
import jax.numpy as jnp
from jax.experimental import pallas as pl
from jax.experimental.pallas import tpu as pltpu

TOKENS = 8192
D_MODEL = 4096
D_HID = 1024
N_EXPERTS = 64

BLK_M = 512
N_BLK = TOKENS // BLK_M
HID_CHUNK = 256
N_CHUNK = D_HID // HID_CHUNK


def _gate_kernel(x_ref, w1_ref, b1_ref, w2_ref, b2_ref, out_ref, lg2):
    i = pl.program_id(0)

    x = x_ref[...]
    logits = jnp.broadcast_to(b2_ref[...], (BLK_M, N_EXPERTS))
    for n in range(N_CHUNK):
        cols = pl.ds(n * HID_CHUNK, HID_CHUNK)
        h = jnp.dot(x, w1_ref[:, cols], preferred_element_type=jnp.float32)
        h = jnp.maximum(h + b1_ref[:, cols], 0.0)
        logits = logits + jnp.dot(h, w2_ref[cols, :],
                                  preferred_element_type=jnp.float32)
    lg2[i % 2] = logits

    @pl.when(i % 2 == 1)
    def _softmax_out():
        lg = lg2[...].reshape(2 * BLK_M, N_EXPERTS)
        m = jnp.max(lg, axis=-1, keepdims=True)
        e = jnp.exp(lg - m)
        out_ref[...] = e / jnp.sum(e, axis=-1, keepdims=True)


@jax.jit
def kernel(x, W1, b1, W2, b2):
    b1_2d = b1.reshape(1, D_HID)
    b2_2d = b2.reshape(1, N_EXPERTS)
    grid = (N_BLK,)
    return pl.pallas_call(
        _gate_kernel,
        grid=grid,
        in_specs=[
            pl.BlockSpec((BLK_M, D_MODEL), lambda i: (i, 0)),
            pl.BlockSpec((D_MODEL, D_HID), lambda i: (0, 0)),
            pl.BlockSpec((1, D_HID), lambda i: (0, 0)),
            pl.BlockSpec((D_HID, N_EXPERTS), lambda i: (0, 0)),
            pl.BlockSpec((1, N_EXPERTS), lambda i: (0, 0)),
        ],
        out_specs=pl.BlockSpec((2 * BLK_M, N_EXPERTS), lambda i: (i // 2, 0)),
        out_shape=jax.ShapeDtypeStruct((TOKENS, N_EXPERTS), jnp.float32),
        scratch_shapes=[pltpu.VMEM((2, BLK_M, N_EXPERTS), jnp.float32)],
    )(x, W1, b1_2d, W2, b2_2d)


# speedup vs baseline: 1.3855x; 1.3855x over previous
import jax
import jax.numpy as jnp
from jax.experimental import pallas as pl
from jax.experimental.pallas import tpu as pltpu

TOKENS = 8192
D_MODEL = 4096
D_HID = 1024
N_EXPERTS = 64

BLK_M = 512
N_BLK = TOKENS // BLK_M


def _gate_kernel(x_ref, w1_ref, b1_ref, w2_ref, b2_ref, out_ref, h2):
    i = pl.program_id(0)

    h = jnp.dot(x_ref[...], w1_ref[...], preferred_element_type=jnp.float32)

    h_prev = h2[(i + 1) % 2]
    logits = jnp.dot(h_prev, w2_ref[...],
                     preferred_element_type=jnp.float32) + b2_ref[...]
    m = jnp.max(logits, axis=-1, keepdims=True)
    e = jnp.exp(logits - m)
    inv = 1.0 / jnp.sum(e, axis=-1, keepdims=True)
    out_ref[...] = e * inv

    h2[i % 2] = jnp.maximum(h + b1_ref[...], 0.0)


@jax.jit
def kernel(x, W1, b1, W2, b2):
    b1_2d = b1.reshape(1, D_HID)
    b2_2d = b2.reshape(1, N_EXPERTS)
    grid = (N_BLK + 1,)
    return pl.pallas_call(
        _gate_kernel,
        grid=grid,
        in_specs=[
            pl.BlockSpec((BLK_M, D_MODEL),
                         lambda i: (jnp.minimum(i, N_BLK - 1), 0)),
            pl.BlockSpec((D_MODEL, D_HID), lambda i: (0, 0)),
            pl.BlockSpec((1, D_HID), lambda i: (0, 0)),
            pl.BlockSpec((D_HID, N_EXPERTS), lambda i: (0, 0)),
            pl.BlockSpec((1, N_EXPERTS), lambda i: (0, 0)),
        ],
        out_specs=pl.BlockSpec((BLK_M, N_EXPERTS),
                               lambda i: (jnp.maximum(i - 1, 0), 0)),
        out_shape=jax.ShapeDtypeStruct((TOKENS, N_EXPERTS), jnp.float32),
        scratch_shapes=[pltpu.VMEM((2, BLK_M, D_HID), jnp.float32)],
    )(x, W1, b1_2d, W2, b2_2d)


# speedup vs baseline: 1.4123x; 1.0193x over previous
import jax
import jax.numpy as jnp
from jax.experimental import pallas as pl
from jax.experimental.pallas import tpu as pltpu

TOKENS = 8192
D_MODEL = 4096
D_HID = 1024
N_EXPERTS = 64

BLK_M = 512
N_BLK = TOKENS // BLK_M


def _softmax(logits):
    m = jnp.max(logits, axis=-1, keepdims=True)
    e = jnp.exp(logits - m)
    return e * (1.0 / jnp.sum(e, axis=-1, keepdims=True))


def _gate_kernel(x_ref, w1_ref, b1_ref, w2_ref, b2_ref, out_ref, last_ref,
                 h2):
    i = pl.program_id(0)

    h = jnp.dot(x_ref[...], w1_ref[...], preferred_element_type=jnp.float32)

    h_prev = h2[(i + 1) % 2]
    logits = jnp.dot(h_prev, w2_ref[...],
                     preferred_element_type=jnp.float32) + b2_ref[...]
    out_ref[...] = _softmax(logits)

    h_cur = jnp.maximum(h + b1_ref[...], 0.0)
    h2[i % 2] = h_cur

    @pl.when(i == N_BLK - 1)
    def _last_tail():
        lg = jnp.dot(h_cur, w2_ref[...],
                     preferred_element_type=jnp.float32) + b2_ref[...]
        last_ref[...] = _softmax(lg)


@jax.jit
def kernel(x, W1, b1, W2, b2):
    b1_2d = b1.reshape(1, D_HID)
    b2_2d = b2.reshape(1, N_EXPERTS)
    grid = (N_BLK,)
    out, last = pl.pallas_call(
        _gate_kernel,
        grid=grid,
        in_specs=[
            pl.BlockSpec((BLK_M, D_MODEL), lambda i: (i, 0)),
            pl.BlockSpec((D_MODEL, D_HID), lambda i: (0, 0)),
            pl.BlockSpec((1, D_HID), lambda i: (0, 0)),
            pl.BlockSpec((D_HID, N_EXPERTS), lambda i: (0, 0)),
            pl.BlockSpec((1, N_EXPERTS), lambda i: (0, 0)),
        ],
        out_specs=[
            pl.BlockSpec((BLK_M, N_EXPERTS),
                         lambda i: (jnp.maximum(i - 1, 0), 0)),
            pl.BlockSpec((BLK_M, N_EXPERTS), lambda i: (0, 0)),
        ],
        out_shape=[
            jax.ShapeDtypeStruct((TOKENS, N_EXPERTS), jnp.float32),
            jax.ShapeDtypeStruct((BLK_M, N_EXPERTS), jnp.float32),
        ],
        scratch_shapes=[pltpu.VMEM((2, BLK_M, D_HID), jnp.float32)],
    )(x, W1, b1_2d, W2, b2_2d)
    return jax.lax.dynamic_update_slice(out, last, (TOKENS - BLK_M, 0))
